# baseline (speedup 1.0000x reference)
import functools

import jax
import jax.numpy as jnp
from jax import lax
from jax.experimental import pallas as pl
from jax.experimental.pallas import tpu as pltpu
from jax.experimental.pallas import tpu_sc as plsc

_B = 4096
_L = 200
_D = 32
_NTOK = _B * _L
_SROWS = 8
_STOK = _SROWS * _L
_K = 4
_BK = _B // _K
_NSTRIPE = _BK // _SROWS

_info = plsc.get_sparse_core_info()
_NC = _info.num_cores
_NS = _info.num_subcores
_NW = _NC * _NS
_SPW = _NSTRIPE // _NW
_NVREG = _STOK // 16

_mesh = plsc.VectorSubcoreMesh(core_axis_name="c", subcore_axis_name="s")


@functools.partial(
    pl.kernel,
    mesh=_mesh,
    out_type=jax.ShapeDtypeStruct((_BK * _L, _D), jnp.float32),
    scratch_types=[
        pltpu.VMEM((_STOK,), jnp.int32),
        pltpu.VMEM((_STOK,), jnp.int32),
        pltpu.VMEM((_STOK,), jnp.int32),
        pltpu.VMEM((_STOK,), jnp.int32),
        pltpu.VMEM((_STOK, _D), jnp.float32),
        pltpu.VMEM((_STOK, _D), jnp.float32),
        pltpu.SemaphoreType.DMA,
        pltpu.SemaphoreType.DMA,
        pltpu.SemaphoreType.DMA,
        pltpu.SemaphoreType.DMA,
    ],
    compiler_params=pltpu.CompilerParams(
        use_tc_tiling_on_sc=False, needs_layout_passes=False
    ),
)
def _emb_gather(idx_hbm, table_hbm, out_hbm,
                raw0, raw1, ord0, ord1, rows0, rows1, g0, g1, w0, w1):
    wid = lax.axis_index("s") * _NC + lax.axis_index("c")
    base = wid * _SPW
    raw = (raw0, raw1)
    order = (ord0, ord1)
    rows = (rows0, rows1)
    gsem = (g0, g1)
    wsem = (w0, w1)

    iota = lax.iota(jnp.int32, 16)
    pattern = ((iota >> 2) & 3) * _L + (iota & 3)

    def stage_stripe(i, b):
        s = base + i
        pltpu.sync_copy(idx_hbm.at[pl.ds(s * _STOK, _STOK)], raw[b])
        for v in range(_NVREG):
            k0 = 16 * v
            a = (k0 >> 5) * 4 + ((k0 >> 2) & 7) * _L
            src = pattern + a
            order[b][pl.ds(k0, 16)] = plsc.load_gather(raw[b], [src])
        pltpu.async_copy(table_hbm.at[order[b]], rows[b], gsem[b])

    stage_stripe(0, 0)
    stage_stripe(1, 1)

    def body(i0):
        for b in range(2):
            i = i0 + b
            pltpu.make_async_copy(table_hbm.at[order[b]], rows[b], gsem[b]).wait()
            pltpu.async_copy(rows[b],
                             out_hbm.at[pl.ds((base + i) * _STOK, _STOK)],
                             wsem[b])

            @pl.when(i < _SPW - 2)
            def _():
                pltpu.make_async_copy(
                    rows[b], out_hbm.at[pl.ds(base, _STOK)], wsem[b]).wait()
                stage_stripe(i + 2, b)

    pl.loop(0, _SPW, step=2)(body)

    pltpu.make_async_copy(rows[0], out_hbm.at[pl.ds(base, _STOK)], w0).wait()
    pltpu.make_async_copy(rows[1], out_hbm.at[pl.ds(base, _STOK)], w1).wait()


def kernel(sequence, emb_weight):
    idx = sequence.reshape(-1).astype(jnp.int32)
    npart = _BK * _L
    res = jnp.zeros((_B, _L * _D), jnp.float32)
    for k in range(_K):
        idx_k = lax.slice(idx, (k * npart,), ((k + 1) * npart,))
        out_k = _emb_gather(idx_k, emb_weight)
        part = (
            out_k.reshape(_BK // 8, _L // 4, 8, 4 * _D)
            .transpose(0, 2, 1, 3)
            .reshape(_BK, _L * _D)
        )
        res = lax.dynamic_update_slice(res, part, (k * _BK, 0))
    return res


# speedup vs baseline: 1.1428x; 1.1428x over previous
import functools

import jax
import jax.numpy as jnp
from jax import lax
from jax.experimental import pallas as pl
from jax.experimental.pallas import tpu as pltpu
from jax.experimental.pallas import tpu_sc as plsc

_B = 4096
_L = 200
_D = 32
_NTOK = _B * _L
_SROWS = 8
_STOK = _SROWS * _L
_NSTRIPE = _B // _SROWS

_info = plsc.get_sparse_core_info()
_NC = _info.num_cores
_NS = _info.num_subcores
_NW = _NC * _NS
_SPW = _NSTRIPE // _NW
_NVREG = _STOK // 16

_mesh = plsc.VectorSubcoreMesh(core_axis_name="c", subcore_axis_name="s")


@functools.partial(
    pl.kernel,
    mesh=_mesh,
    out_type=jax.ShapeDtypeStruct((_NTOK, _D), jnp.float32),
    scratch_types=[
        pltpu.VMEM((_STOK,), jnp.int32),
        pltpu.VMEM((_STOK,), jnp.int32),
        pltpu.VMEM((_STOK,), jnp.int32),
        pltpu.VMEM((_STOK,), jnp.int32),
        pltpu.VMEM((_STOK, _D), jnp.float32),
        pltpu.VMEM((_STOK, _D), jnp.float32),
        pltpu.SemaphoreType.DMA,
        pltpu.SemaphoreType.DMA,
        pltpu.SemaphoreType.DMA,
        pltpu.SemaphoreType.DMA,
    ],
    compiler_params=pltpu.CompilerParams(
        use_tc_tiling_on_sc=False, needs_layout_passes=False
    ),
)
def _emb_gather(idx_hbm, table_hbm, out_hbm,
                raw0, raw1, ord0, ord1, rows0, rows1, g0, g1, w0, w1):
    wid = lax.axis_index("s") * _NC + lax.axis_index("c")
    base = wid * _SPW
    raw = (raw0, raw1)
    order = (ord0, ord1)
    rows = (rows0, rows1)
    gsem = (g0, g1)
    wsem = (w0, w1)

    iota = lax.iota(jnp.int32, 16)
    pattern = ((iota >> 2) & 3) * _L + (iota & 3)

    def stage_stripe(i, b):
        s = base + i
        pltpu.sync_copy(idx_hbm.at[pl.ds(s * _STOK, _STOK)], raw[b])
        for v in range(_NVREG):
            k0 = 16 * v
            a = (k0 >> 5) * 4 + ((k0 >> 2) & 7) * _L
            src = pattern + a
            order[b][pl.ds(k0, 16)] = plsc.load_gather(raw[b], [src])
        pltpu.async_copy(table_hbm.at[order[b]], rows[b], gsem[b])

    stage_stripe(0, 0)
    stage_stripe(1, 1)

    def body(i0):
        for b in range(2):
            i = i0 + b
            pltpu.make_async_copy(table_hbm.at[order[b]], rows[b], gsem[b]).wait()
            pltpu.async_copy(rows[b],
                             out_hbm.at[pl.ds((base + i) * _STOK, _STOK)],
                             wsem[b])

            @pl.when(i < _SPW - 2)
            def _():
                pltpu.make_async_copy(
                    rows[b], out_hbm.at[pl.ds(base, _STOK)], wsem[b]).wait()
                stage_stripe(i + 2, b)

    pl.loop(0, _SPW, step=2)(body)

    pltpu.make_async_copy(rows[0], out_hbm.at[pl.ds(base, _STOK)], w0).wait()
    pltpu.make_async_copy(rows[1], out_hbm.at[pl.ds(base, _STOK)], w1).wait()


def kernel(sequence, emb_weight):
    idx = sequence.reshape(-1).astype(jnp.int32)
    out = _emb_gather(idx, emb_weight)
    return (
        out.reshape(_B // 8, _L // 4, 8, 4 * _D)
        .transpose(0, 2, 1, 3)
        .reshape(_B, _L * _D)
    )
